# baseline (speedup 1.0000x reference)
import functools

import jax
import jax.numpy as jnp
from jax import lax
from jax.experimental import pallas as pl
from jax.experimental.pallas import tpu as pltpu
from jax.experimental.pallas import tpu_sc as plsc

NUM_CORES = 2
NUM_SUBCORES = 16
LANES = 16
CHUNK = 112
NBUF = 2


def _sc_gather_scale_scatter(h, packed, w):
    N, D = h.shape
    E = packed.shape[0]
    nw = NUM_CORES * NUM_SUBCORES
    nch = E // (CHUNK * nw)
    assert E == nch * CHUNK * nw and nch % (2 * NBUF) == 0 and nch >= 4
    assert N <= 65536
    rows_per_tile = (N // (8 * NUM_SUBCORES)) * 8
    leftover = N - rows_per_tile * NUM_SUBCORES
    n_full, rem = divmod(rows_per_tile, CHUNK)
    assert leftover % 8 == 0 and leftover <= CHUNK and rem % 8 == 0

    mesh = plsc.VectorSubcoreMesh(
        core_axis_name="c", subcore_axis_name="s",
        num_cores=NUM_CORES, num_subcores=NUM_SUBCORES)

    @functools.partial(
        pl.kernel,
        out_type=jax.ShapeDtypeStruct((NUM_CORES, N, D), jnp.float32),
        mesh=mesh,
        scratch_types=[
            pltpu.VMEM((nch * CHUNK,), jnp.int32),
            pltpu.VMEM((nch * CHUNK,), jnp.float32),
            pltpu.VMEM((2 * NBUF, CHUNK), jnp.int32),
            pltpu.VMEM((2 * NBUF, CHUNK), jnp.int32),
            [pltpu.VMEM((CHUNK, D), jnp.float32) for _ in range(NBUF)],
            pltpu.VMEM_SHARED((N, D), jnp.float32),
            [pltpu.SemaphoreType.DMA for _ in range(NBUF)],
            [pltpu.SemaphoreType.DMA for _ in range(NBUF)],
        ],
    )
    def sck(h_hbm, pk_hbm, w_hbm, out_hbm,
            pk_t, w_t, src_r, dst_r, rows, agg_sh, gsem, ssem):
        c = lax.axis_index("c")
        s = lax.axis_index("s")
        wid = s * NUM_CORES + c
        ebase = wid * nch * CHUNK
        zero16 = jnp.zeros((LANES,), jnp.float32)
        r0 = s * rows_per_tile
        rext = rows_per_tile * NUM_SUBCORES

        pltpu.sync_copy(pk_hbm.at[pl.ds(ebase, nch * CHUNK)], pk_t)
        pltpu.sync_copy(w_hbm.at[pl.ds(ebase, nch * CHUNK)], w_t)

        def unpack(j, m):
            for g in range(CHUNK // LANES):
                pd = pk_t[pl.ds(j * CHUNK + g * LANES, LANES)]
                src_r[m, pl.ds(g * LANES, LANES)] = pd & 0xFFFF
                dst_r[m, pl.ds(g * LANES, LANES)] = (
                    lax.shift_right_logical(pd, 16))

        def scale(buf, j):
            def egroup(g, inner):
                wg = w_t[pl.ds(j * CHUNK + g * LANES, LANES)]
                for i in range(LANES):
                    e = g * LANES + i
                    wb = jnp.full((LANES,), wg[i])
                    for k in range(D // LANES):
                        sl = pl.ds(k * LANES, LANES)
                        buf[e, sl] = buf[e, sl] * wb
                return inner
            lax.fori_loop(0, CHUNK // LANES, egroup, 0)

        def zrow(i, carry):
            for k in range(D // LANES):
                rows[0][i, pl.ds(k * LANES, LANES)] = zero16
            return carry
        lax.fori_loop(0, CHUNK, zrow, 0)

        for j in range(n_full):
            pltpu.sync_copy(rows[0].at[pl.ds(0, CHUNK)],
                            agg_sh.at[pl.ds(r0 + j * CHUNK, CHUNK)])
        if rem:
            pltpu.sync_copy(rows[0].at[pl.ds(0, rem)],
                            agg_sh.at[pl.ds(r0 + n_full * CHUNK, rem)])
        if leftover:
            @pl.when(s == 0)
            def _():
                pltpu.sync_copy(rows[0].at[pl.ds(0, leftover)],
                                agg_sh.at[pl.ds(rext, leftover)])

        for jj in range(NBUF):
            unpack(jj, jj)
            pltpu.async_copy(h_hbm.at[src_r.at[jj]], rows[jj], gsem[jj])
        plsc.subcore_barrier()

        def body(it, carry):
            for u in range(2 * NBUF):
                j = it * 2 * NBUF + u
                b = u % NBUF
                m = u
                m2 = (u + NBUF) % (2 * NBUF)
                pltpu.make_async_copy(h_hbm.at[src_r.at[m]], rows[b],
                                      gsem[b]).wait()
                scale(rows[b], j)
                pltpu.async_copy(rows[b], agg_sh.at[dst_r.at[m]],
                                 ssem[b], add=True)

                @pl.when(j + NBUF < nch)
                def _():
                    unpack(j + NBUF, m2)
                pltpu.make_async_copy(rows[b], agg_sh.at[dst_r.at[m]],
                                      ssem[b]).wait()

                @pl.when(j + NBUF < nch)
                def _():
                    pltpu.async_copy(h_hbm.at[src_r.at[m2]], rows[b],
                                     gsem[b])
            return carry
        lax.fori_loop(0, nch // (2 * NBUF), body, 0)
        plsc.subcore_barrier()

        for j in range(n_full):
            sl = pl.ds(r0 + j * CHUNK, CHUNK)
            pltpu.sync_copy(agg_sh.at[sl], out_hbm.at[c, sl])
        if rem:
            sl = pl.ds(r0 + n_full * CHUNK, rem)
            pltpu.sync_copy(agg_sh.at[sl], out_hbm.at[c, sl])
        if leftover:
            @pl.when(s == 0)
            def _():
                sl = pl.ds(rext, leftover)
                pltpu.sync_copy(agg_sh.at[sl], out_hbm.at[c, sl])

    return sck(h, packed, w)


def _tc_linear(h, p0, p1, W, b, relu):
    N, D = h.shape
    blk = 1000
    assert N % blk == 0

    def body(h_ref, p0_ref, p1_ref, w_ref, b_ref, o_ref):
        rst = h_ref[...] + p0_ref[...] + p1_ref[...]
        acc = jnp.dot(rst, w_ref[...],
                      preferred_element_type=jnp.float32) + b_ref[...]
        o_ref[...] = jnp.maximum(acc, 0.0) if relu else acc

    return pl.pallas_call(
        body,
        grid=(N // blk,),
        in_specs=[
            pl.BlockSpec((blk, D), lambda i: (i, 0)),
            pl.BlockSpec((blk, D), lambda i: (i, 0)),
            pl.BlockSpec((blk, D), lambda i: (i, 0)),
            pl.BlockSpec((D, D), lambda i: (0, 0)),
            pl.BlockSpec((1, D), lambda i: (0, 0)),
        ],
        out_specs=pl.BlockSpec((blk, D), lambda i: (i, 0)),
        out_shape=jax.ShapeDtypeStruct((N, D), jnp.float32),
    )(h, p0, p1, W, b.reshape(1, D))


def kernel(x, edge_index, edge_attr, W0, b0, W1, b1, W2, b2):
    src = edge_index[0]
    dst = edge_index[1]
    w = edge_attr
    E = src.shape[0]
    unit = CHUNK * NUM_CORES * NUM_SUBCORES * 2 * NBUF
    e_pad = (unit - E % unit) % unit
    if e_pad:
        src = jnp.pad(src, (0, e_pad))
        dst = jnp.pad(dst, (0, e_pad))
        w = jnp.pad(w, (0, e_pad))
    packed = src | (dst << 16)

    h = x
    for i, (W, b) in enumerate(((W0, b0), (W1, b1), (W2, b2))):
        parts = _sc_gather_scale_scatter(h, packed, w)
        h = _tc_linear(h, parts[0], parts[1], W, b, relu=(i < 2))
    return h


# speedup vs baseline: 2.2191x; 2.2191x over previous
import functools

import jax
import jax.numpy as jnp
from jax import lax
from jax.experimental import pallas as pl
from jax.experimental.pallas import tpu as pltpu
from jax.experimental.pallas import tpu_sc as plsc

NUM_CORES = 2
NUM_SUBCORES = 16
LANES = 16
CHUNK = 112
NBUF = 2


def _sc_gather_scale_scatter(h, packed, w):
    N, D = h.shape
    E = packed.shape[0]
    nw = NUM_CORES * NUM_SUBCORES
    nch = E // (CHUNK * nw)
    assert E == nch * CHUNK * nw and nch % NBUF == 0 and nch >= 4
    assert N <= 65536
    rows_per_tile = (N // (8 * NUM_SUBCORES)) * 8
    leftover = N - rows_per_tile * NUM_SUBCORES
    n_full, rem = divmod(rows_per_tile, CHUNK)
    assert leftover % 8 == 0 and leftover <= CHUNK and rem % 8 == 0

    mesh = plsc.VectorSubcoreMesh(
        core_axis_name="c", subcore_axis_name="s",
        num_cores=NUM_CORES, num_subcores=NUM_SUBCORES)

    @functools.partial(
        pl.kernel,
        out_type=jax.ShapeDtypeStruct((NUM_CORES, N, D), jnp.float32),
        mesh=mesh,
        scratch_types=[
            pltpu.VMEM((nch * CHUNK,), jnp.int32),
            pltpu.VMEM((nch * CHUNK,), jnp.float32),
            pltpu.VMEM((NBUF, CHUNK), jnp.int32),
            pltpu.VMEM((NBUF, CHUNK), jnp.int32),
            [pltpu.VMEM((CHUNK, D), jnp.float32) for _ in range(NBUF)],
            pltpu.VMEM_SHARED((N, D), jnp.float32),
            [pltpu.SemaphoreType.DMA for _ in range(NBUF)],
            [pltpu.SemaphoreType.DMA for _ in range(NBUF)],
        ],
    )
    def sck(h_hbm, pk_hbm, w_hbm, out_hbm,
            pk_t, w_t, src_r, dst_r, rows, agg_sh, gsem, ssem):
        c = lax.axis_index("c")
        s = lax.axis_index("s")
        wid = s * NUM_CORES + c
        ebase = wid * nch * CHUNK
        zero16 = jnp.zeros((LANES,), jnp.float32)
        r0 = s * rows_per_tile
        rext = rows_per_tile * NUM_SUBCORES

        pltpu.sync_copy(pk_hbm.at[pl.ds(ebase, nch * CHUNK)], pk_t)
        pltpu.sync_copy(w_hbm.at[pl.ds(ebase, nch * CHUNK)], w_t)

        def unpack(j, m):
            for g in range(CHUNK // LANES):
                pd = pk_t[pl.ds(j * CHUNK + g * LANES, LANES)]
                src_r[m, pl.ds(g * LANES, LANES)] = pd & 0xFFFF
                dst_r[m, pl.ds(g * LANES, LANES)] = (
                    lax.shift_right_logical(pd, 16))

        def scale(buf, j):
            def egroup(g, inner):
                wg = w_t[pl.ds(j * CHUNK + g * LANES, LANES)]
                for i in range(LANES):
                    e = g * LANES + i
                    wb = jnp.full((LANES,), wg[i])
                    for k in range(D // LANES):
                        sl = pl.ds(k * LANES, LANES)
                        buf[e, sl] = buf[e, sl] * wb
                return inner
            lax.fori_loop(0, CHUNK // LANES, egroup, 0)

        def zrow(i, carry):
            for k in range(D // LANES):
                rows[0][i, pl.ds(k * LANES, LANES)] = zero16
            return carry
        lax.fori_loop(0, CHUNK, zrow, 0)

        for j in range(n_full):
            pltpu.sync_copy(rows[0].at[pl.ds(0, CHUNK)],
                            agg_sh.at[pl.ds(r0 + j * CHUNK, CHUNK)])
        if rem:
            pltpu.sync_copy(rows[0].at[pl.ds(0, rem)],
                            agg_sh.at[pl.ds(r0 + n_full * CHUNK, rem)])
        if leftover:
            @pl.when(s == 0)
            def _():
                pltpu.sync_copy(rows[0].at[pl.ds(0, leftover)],
                                agg_sh.at[pl.ds(rext, leftover)])

        for jj in range(NBUF):
            unpack(jj, jj)
            pltpu.async_copy(h_hbm.at[src_r.at[jj]], rows[jj], gsem[jj])
        plsc.subcore_barrier()

        def body(it, carry):
            for b in range(NBUF):
                j = it * NBUF + b
                pltpu.make_async_copy(h_hbm.at[src_r.at[b]], rows[b],
                                      gsem[b]).wait()
                scale(rows[b], j)
                pltpu.async_copy(rows[b], agg_sh.at[dst_r.at[b]],
                                 ssem[b], add=True)
                pltpu.make_async_copy(rows[b], agg_sh.at[dst_r.at[b]],
                                      ssem[b]).wait()

                @pl.when(j + NBUF < nch)
                def _():
                    unpack(j + NBUF, b)
                    pltpu.async_copy(h_hbm.at[src_r.at[b]], rows[b],
                                     gsem[b])
            return carry
        lax.fori_loop(0, nch // NBUF, body, 0)
        plsc.subcore_barrier()

        for j in range(n_full):
            sl = pl.ds(r0 + j * CHUNK, CHUNK)
            pltpu.sync_copy(agg_sh.at[sl], out_hbm.at[c, sl])
        if rem:
            sl = pl.ds(r0 + n_full * CHUNK, rem)
            pltpu.sync_copy(agg_sh.at[sl], out_hbm.at[c, sl])
        if leftover:
            @pl.when(s == 0)
            def _():
                sl = pl.ds(rext, leftover)
                pltpu.sync_copy(agg_sh.at[sl], out_hbm.at[c, sl])

    return sck(h, packed, w)


def _tc_linear(h, p0, p1, W, b, relu):
    N, D = h.shape
    blk = 1000
    assert N % blk == 0

    def body(h_ref, p0_ref, p1_ref, w_ref, b_ref, o_ref):
        rst = h_ref[...] + p0_ref[...] + p1_ref[...]
        acc = jnp.dot(rst, w_ref[...],
                      preferred_element_type=jnp.float32) + b_ref[...]
        o_ref[...] = jnp.maximum(acc, 0.0) if relu else acc

    return pl.pallas_call(
        body,
        grid=(N // blk,),
        in_specs=[
            pl.BlockSpec((blk, D), lambda i: (i, 0)),
            pl.BlockSpec((blk, D), lambda i: (i, 0)),
            pl.BlockSpec((blk, D), lambda i: (i, 0)),
            pl.BlockSpec((D, D), lambda i: (0, 0)),
            pl.BlockSpec((1, D), lambda i: (0, 0)),
        ],
        out_specs=pl.BlockSpec((blk, D), lambda i: (i, 0)),
        out_shape=jax.ShapeDtypeStruct((N, D), jnp.float32),
    )(h, p0, p1, W, b.reshape(1, D))


def kernel(x, edge_index, edge_attr, W0, b0, W1, b1, W2, b2):
    src = edge_index[0]
    dst = edge_index[1]
    w = edge_attr
    E = src.shape[0]
    unit = CHUNK * NUM_CORES * NUM_SUBCORES * NBUF
    e_pad = (unit - E % unit) % unit
    if e_pad:
        src = jnp.pad(src, (0, e_pad))
        dst = jnp.pad(dst, (0, e_pad))
        w = jnp.pad(w, (0, e_pad))
    packed = src | (dst << 16)

    h = x
    for i, (W, b) in enumerate(((W0, b0), (W1, b1), (W2, b2))):
        parts = _sc_gather_scale_scatter(h, packed, w)
        h = _tc_linear(h, parts[0], parts[1], W, b, relu=(i < 2))
    return h
